# baseline (speedup 1.0000x reference)
import jax
import jax.numpy as jnp
from jax import lax
from jax.experimental import pallas as pl
from jax.experimental.pallas import tpu as pltpu
from jax.experimental.pallas import tpu_sc as plsc

_N = 1_000_000
_D = 64
_NC = 2
_NS = 16
_NW = _NC * _NS
_STEP = 1000
_NSTEP = _N // _STEP


def _sc_body(x_hbm, o_hbm, buf):
    wid = lax.axis_index("s") * _NC + lax.axis_index("c")
    nsteps_w = (_NSTEP - 1 - wid) // _NW + 1

    def step(j, carry):
        start = (wid + j * _NW) * _STEP
        pltpu.sync_copy(x_hbm.at[pl.ds(start, _STEP), :], buf)

        @pl.when(jnp.logical_and(wid == 0, j == 0))
        def _():
            two = jnp.full((16,), 2.0, jnp.float32)
            for c in range(_D // 16):
                buf[1, pl.ds(c * 16, 16)] = two
                buf[3, pl.ds(c * 16, 16)] = two

        pltpu.sync_copy(buf, o_hbm.at[pl.ds(start, _STEP), :])
        return carry

    lax.fori_loop(0, nsteps_w, step, 0)


def kernel(x):
    f = pl.kernel(
        _sc_body,
        out_type=jax.ShapeDtypeStruct((_N, _D), jnp.float32),
        mesh=plsc.VectorSubcoreMesh(core_axis_name="c", subcore_axis_name="s"),
        scratch_types=[pltpu.VMEM((_STEP, _D), jnp.float32)],
    )
    return f(x)
